# baseline (speedup 1.0000x reference)
import functools

import jax
import jax.numpy as jnp
from jax import lax
from jax.experimental import pallas as pl
from jax.experimental.pallas import tpu as pltpu, tpu_sc as plsc

NC, NS, L = 2, 16, 16
NW = NC * NS

B = 16384
D = 128
BPW = B // NW
C = 128
NCHUNK = BPW // C
G = C // L


def _mf_body(users, items, ut, it, out, uidx, iidx, urows0, irows0, urows1,
             irows1, outbuf, sem_u0, sem_i0, sem_u1, sem_i1):
    wid = lax.axis_index("s") * NC + lax.axis_index("c")
    base = wid * BPW
    pltpu.sync_copy(users.at[pl.ds(base, BPW)], uidx)
    pltpu.sync_copy(items.at[pl.ds(base, BPW)], iidx)
    lane = lax.iota(jnp.int32, 16)

    ubufs = (urows0, urows1)
    ibufs = (irows0, irows1)
    usems = (sem_u0, sem_u1)
    isems = (sem_i0, sem_i1)

    def start(c):
        b = c % 2
        cu = pltpu.async_copy(ut.at[uidx.at[pl.ds(c * C, C)]], ubufs[b],
                              usems[b])
        ci = pltpu.async_copy(it.at[iidx.at[pl.ds(c * C, C)]], ibufs[b],
                              isems[b])
        return cu, ci

    inflight = [start(0), start(1)]

    for c in range(NCHUNK):
        cu, ci = inflight[c]
        cu.wait()
        ci.wait()
        urows, irows = ubufs[c % 2], ibufs[c % 2]

        @pl.loop(0, G)
        def _grp(g, c=c, urows=urows, irows=irows):
            res = jnp.zeros((L,), jnp.float32)
            for rr in range(L):
                r = g * L + rr
                acc = jnp.zeros((L,), jnp.float32)
                for k in range(D // L):
                    acc = acc + urows[r, pl.ds(k * L, L)] * irows[r, pl.ds(k * L, L)]
                s = jnp.sum(acc)
                res = jnp.where(lane == rr, s, res)
            outbuf[pl.ds(c * C + g * L, L)] = res * (1.0 / D)

        if c + 2 < NCHUNK:
            inflight.append(start(c + 2))

    pltpu.sync_copy(outbuf, out.at[pl.ds(base, BPW)])


@jax.jit
def kernel(users, items, user_table, item_table):
    mesh = plsc.VectorSubcoreMesh(core_axis_name="c", subcore_axis_name="s")
    f = pl.kernel(
        _mf_body,
        out_type=jax.ShapeDtypeStruct((B,), jnp.float32),
        mesh=mesh,
        compiler_params=pltpu.CompilerParams(needs_layout_passes=False),
        scratch_types=[
            pltpu.VMEM((BPW,), jnp.int32),
            pltpu.VMEM((BPW,), jnp.int32),
            pltpu.VMEM((C, D), jnp.float32),
            pltpu.VMEM((C, D), jnp.float32),
            pltpu.VMEM((C, D), jnp.float32),
            pltpu.VMEM((C, D), jnp.float32),
            pltpu.VMEM((BPW,), jnp.float32),
            pltpu.SemaphoreType.DMA,
            pltpu.SemaphoreType.DMA,
            pltpu.SemaphoreType.DMA,
            pltpu.SemaphoreType.DMA,
        ],
    )
    return f(users.astype(jnp.int32), items.astype(jnp.int32),
             user_table, item_table)


# speedup vs baseline: 1.1477x; 1.1477x over previous
import functools

import jax
import jax.numpy as jnp
from jax import lax
from jax.experimental import pallas as pl
from jax.experimental.pallas import tpu as pltpu, tpu_sc as plsc

NC, NS, L = 2, 16, 16
NW = NC * NS

B = 16384
D = 128
BPW = B // NW
C = 128
NCHUNK = BPW // C
G = C // L


def _mf_body(users, items, ut, it, out, uidx, iidx, urows, irows, outbuf,
             sem_u, sem_i):
    wid = lax.axis_index("s") * NC + lax.axis_index("c")
    base = wid * BPW
    pltpu.sync_copy(users.at[pl.ds(base, BPW)], uidx)
    pltpu.sync_copy(items.at[pl.ds(base, BPW)], iidx)
    lane = lax.iota(jnp.int32, 16)

    def start(c, b):
        pltpu.async_copy(ut.at[uidx.at[pl.ds(c * C, C)]], urows.at[b],
                         sem_u.at[b])
        pltpu.async_copy(it.at[iidx.at[pl.ds(c * C, C)]], irows.at[b],
                         sem_i.at[b])

    start(0, 0)
    start(1, 1)

    @pl.loop(0, NCHUNK)
    def _chunk(c):
        b = lax.rem(c, 2)
        pltpu.make_async_copy(ut.at[uidx.at[pl.ds(c * C, C)]], urows.at[b],
                              sem_u.at[b]).wait()
        pltpu.make_async_copy(it.at[iidx.at[pl.ds(c * C, C)]], irows.at[b],
                              sem_i.at[b]).wait()

        @pl.loop(0, G)
        def _grp(g):
            res = jnp.zeros((L,), jnp.float32)
            for rr in range(L):
                r = g * L + rr
                acc = jnp.zeros((L,), jnp.float32)
                for k in range(D // L):
                    acc = acc + (urows[b, r, pl.ds(k * L, L)]
                                 * irows[b, r, pl.ds(k * L, L)])
                s = jnp.sum(acc)
                res = jnp.where(lane == rr, s, res)
            outbuf[pl.ds(c * C + g * L, L)] = res * (1.0 / D)

        @pl.when(c + 2 < NCHUNK)
        def _():
            start(c + 2, b)

    pltpu.sync_copy(outbuf, out.at[pl.ds(base, BPW)])


@jax.jit
def kernel(users, items, user_table, item_table):
    mesh = plsc.VectorSubcoreMesh(core_axis_name="c", subcore_axis_name="s")
    f = pl.kernel(
        _mf_body,
        out_type=jax.ShapeDtypeStruct((B,), jnp.float32),
        mesh=mesh,
        compiler_params=pltpu.CompilerParams(needs_layout_passes=False),
        scratch_types=[
            pltpu.VMEM((BPW,), jnp.int32),
            pltpu.VMEM((BPW,), jnp.int32),
            pltpu.VMEM((2, C, D), jnp.float32),
            pltpu.VMEM((2, C, D), jnp.float32),
            pltpu.VMEM((BPW,), jnp.float32),
            pltpu.SemaphoreType.DMA((2,)),
            pltpu.SemaphoreType.DMA((2,)),
        ],
    )
    return f(users.astype(jnp.int32), items.astype(jnp.int32),
             user_table, item_table)


# speedup vs baseline: 1.1484x; 1.0006x over previous
import functools

import jax
import jax.numpy as jnp
from jax import lax
from jax.experimental import pallas as pl
from jax.experimental.pallas import tpu as pltpu, tpu_sc as plsc

NC, NS, L = 2, 16, 16
NW = NC * NS

B = 16384
D = 128
BPW = B // NW
C = 128
NCHUNK = BPW // C
G = C // L


def _mf_body(users, items, ut, it, out, uidx, iidx, urows, irows, outbuf,
             sem_u, sem_i):
    wid = lax.axis_index("s") * NC + lax.axis_index("c")
    base = wid * BPW
    pltpu.sync_copy(users.at[pl.ds(base, BPW)], uidx)
    pltpu.sync_copy(items.at[pl.ds(base, BPW)], iidx)
    lane = lax.iota(jnp.int32, 16)

    def start(c, b):
        pltpu.async_copy(ut.at[uidx.at[pl.ds(c * C, C)]], urows.at[b],
                         sem_u.at[b])
        pltpu.async_copy(it.at[iidx.at[pl.ds(c * C, C)]], irows.at[b],
                         sem_i.at[b])

    start(0, 0)
    start(1, 1)

    @pl.loop(0, NCHUNK)
    def _chunk(c):
        b = lax.rem(c, 2)
        with jax.named_scope("dma_wait"):
            pltpu.make_async_copy(ut.at[uidx.at[pl.ds(c * C, C)]], urows.at[b],
                                  sem_u.at[b]).wait()
            pltpu.make_async_copy(it.at[iidx.at[pl.ds(c * C, C)]], irows.at[b],
                                  sem_i.at[b]).wait()

        with jax.named_scope("compute"):
            @pl.loop(0, G)
            def _grp(g):
                res = jnp.zeros((L,), jnp.float32)
                for rr in range(L):
                    r = g * L + rr
                    acc = jnp.zeros((L,), jnp.float32)
                    for k in range(D // L):
                        acc = acc + (urows[b, r, pl.ds(k * L, L)]
                                     * irows[b, r, pl.ds(k * L, L)])
                    s = jnp.sum(acc)
                    res = jnp.where(lane == rr, s, res)
                outbuf[pl.ds(c * C + g * L, L)] = res * (1.0 / D)

        @pl.when(c + 2 < NCHUNK)
        def _():
            start(c + 2, b)

    pltpu.sync_copy(outbuf, out.at[pl.ds(base, BPW)])


@jax.jit
def kernel(users, items, user_table, item_table):
    mesh = plsc.VectorSubcoreMesh(core_axis_name="c", subcore_axis_name="s")
    f = pl.kernel(
        _mf_body,
        out_type=jax.ShapeDtypeStruct((B,), jnp.float32),
        mesh=mesh,
        compiler_params=pltpu.CompilerParams(needs_layout_passes=False),
        scratch_types=[
            pltpu.VMEM((BPW,), jnp.int32),
            pltpu.VMEM((BPW,), jnp.int32),
            pltpu.VMEM((2, C, D), jnp.float32),
            pltpu.VMEM((2, C, D), jnp.float32),
            pltpu.VMEM((BPW,), jnp.float32),
            pltpu.SemaphoreType.DMA((2,)),
            pltpu.SemaphoreType.DMA((2,)),
        ],
    )
    return f(users.astype(jnp.int32), items.astype(jnp.int32),
             user_table, item_table)
